# baseline (speedup 1.0000x reference)
import functools

import jax
import jax.numpy as jnp
from jax import lax
from jax.experimental import pallas as pl
from jax.experimental.pallas import tpu as pltpu
from jax.experimental.pallas import tpu_sc as plsc

N = 10000
H = 256
G = 64
NC = 2
NS = 16
K = 32
CB = 40

ACC_ROWS = 10112


@functools.partial(jax.jit, static_argnames=("dh", "n_chunks"))
def _sc_segsum(h, src_cns, dst_cns, zeros_acc, *, dh, n_chunks):
    mesh = plsc.VectorSubcoreMesh(core_axis_name="c", subcore_axis_name="s")
    zrows = ACC_ROWS // NS

    @functools.partial(
        pl.kernel,
        out_type=jax.ShapeDtypeStruct((NC, ACC_ROWS, dh), jnp.float32),
        mesh=mesh,
        scratch_types=[
            pltpu.VMEM_SHARED((ACC_ROWS, dh), jnp.float32),
            pltpu.VMEM((CB, K), jnp.int32),
            pltpu.VMEM((CB, K), jnp.int32),
            pltpu.VMEM((K, dh), jnp.float32),
            pltpu.VMEM((K, dh), jnp.float32),
        ] + [pltpu.SemaphoreType.DMA] * 4,
    )
    def body(h_hbm, src_hbm, dst_hbm, z_hbm, out_hbm,
             acc, src_idx, dst_idx, r0, r1, sg0, sg1, ss0, ss1):
        rows = (r0, r1)
        sg = (sg0, sg1)
        ss = (ss0, ss1)
        c = lax.axis_index("c")
        s = lax.axis_index("s")

        pltpu.sync_copy(z_hbm.at[pl.ds(s * zrows, zrows)],
                        acc.at[pl.ds(s * zrows, zrows)])
        plsc.subcore_barrier()

        def load_idx(bi):
            pltpu.sync_copy(src_hbm.at[c, s, pl.ds(bi * CB, CB)], src_idx)
            pltpu.sync_copy(dst_hbm.at[c, s, pl.ds(bi * CB, CB)], dst_idx)

        def g(j, t):
            pltpu.async_copy(h_hbm.at[src_idx.at[j]], rows[t], sg[t])

        def wg(j, t):
            pltpu.make_async_copy(
                h_hbm.at[src_idx.at[j]], rows[t], sg[t]).wait()

        def sc(j, t):
            pltpu.async_copy(rows[t], acc.at[dst_idx.at[j]], ss[t], add=True)

        def ws(t):
            pltpu.make_async_copy(
                rows[t], acc.at[dst_idx.at[0]], ss[t]).wait()

        def blk(bi, _):
            load_idx(bi)
            g(0, 0)

            def pair(jj, _):
                j0 = 2 * jj
                j1 = j0 + 1

                @pl.when(jj > 0)
                def _():
                    ws(1)

                g(j1, 1)
                wg(j0, 0)
                sc(j0, 0)

                @pl.when(j1 + 1 < CB)
                def _():
                    ws(0)
                    g(j1 + 1, 0)

                wg(j1, 1)
                sc(j1, 1)
                return 0

            lax.fori_loop(0, CB // 2, pair, 0)
            ws(0)
            ws(1)
            return 0

        lax.fori_loop(0, n_chunks // CB, blk, 0)

        plsc.subcore_barrier()
        pltpu.sync_copy(acc.at[pl.ds(s * zrows, zrows)],
                        out_hbm.at[c, pl.ds(s * zrows, zrows)])

    return body(h, src_cns, dst_cns, zeros_acc)


def _tc_layer(h_in, agg, p, mode):
    def body(h_ref, agg_ref, w1, b1, w2, b2, gamma, beta, out_ref):
        a = agg_ref[...]
        if mode == "edge":
            m = h_ref[...] + a[0, :N] + a[1, :N]
        else:
            m = jnp.concatenate([h_ref[:N] + a[0, :N],
                                 h_ref[N:] + a[1, :N]], axis=1)
        z = jnp.maximum(
            jax.lax.dot_general(m, w1[...], (((1,), (0,)), ((), ())),
                                preferred_element_type=jnp.float32)
            + b1[...], 0.0)
        m2 = jax.lax.dot_general(z, w2[...], (((1,), (0,)), ((), ())),
                                 preferred_element_type=jnp.float32) + b2[...]

        def colmean(v):
            vr = v.reshape(50, 200, H)
            return jnp.sum(jnp.sum(vr, axis=1), axis=0, keepdims=True) / N

        mu = colmean(m2)
        var = colmean((m2 - mu) * (m2 - mu))
        o = jnp.maximum(
            (m2 - mu) * lax.rsqrt(var + 1e-5) * gamma[...] + beta[...], 0.0)
        out_ref[pl.ds(0, N), :] = o[:, : H // 2]
        out_ref[pl.ds(N, N), :] = o[:, H // 2 :]

    return pl.pallas_call(
        body,
        out_shape=jax.ShapeDtypeStruct((2 * N, H // 2), jnp.float32),
    )(h_in, agg, p["W1"], p["b1"][None, :], p["W2"], p["b2"][None, :],
      p["gamma"][None, :], p["beta"][None, :])


def _tc_final_body(hs_ref, bm_ref, gf_ref, wm, wx, wg, bh1, wh2, bh2,
                   out_ref, maxacc):
    h = jnp.concatenate([hs_ref[:N], hs_ref[N:]], axis=1)
    bm = bm_ref[...]
    ids = lax.broadcasted_iota(jnp.int32, (1, G), 1)
    maskf = (bm == ids).astype(jnp.float32)
    counts = jnp.sum(maskf, axis=0, keepdims=True)
    sums = jax.lax.dot_general(maskf, h, (((0,), (0,)), ((), ())),
                               preferred_element_type=jnp.float32)
    mean_pool = sums / jnp.maximum(counts, 1.0).reshape(G, 1)

    def gbody(g, _):
        sel = jnp.where(bm == g, h, -jnp.inf)
        maxacc[pl.ds(g, 1), :] = jnp.max(sel, axis=0, keepdims=True)
        return 0

    lax.fori_loop(0, G, gbody, 0, unroll=False)
    mx = maxacc[...]
    max_pool = jnp.where(jnp.isfinite(mx), mx, 0.0)

    gf = gf_ref[...]
    hid = (jax.lax.dot_general(mean_pool, wm[...], (((1,), (0,)), ((), ())),
                               preferred_element_type=jnp.float32)
           + jax.lax.dot_general(max_pool, wx[...], (((1,), (0,)), ((), ())),
                                 preferred_element_type=jnp.float32)
           + gf[:, 0:1] * wg[0:1, :]
           + gf[:, 1:2] * wg[1:2, :]
           + gf[:, 2:3] * wg[2:3, :]
           + bh1[...])
    hid = jnp.maximum(hid, 0.0)
    out_ref[...] = jax.lax.dot_general(
        hid, wh2[...], (((1,), (0,)), ((), ())),
        preferred_element_type=jnp.float32) + bh2[...]


def _tc_final(hs, batch, gf, params):
    wh1 = params["Wh1"]
    out = pl.pallas_call(
        _tc_final_body,
        out_shape=jax.ShapeDtypeStruct((G, 1), jnp.float32),
        scratch_shapes=[pltpu.VMEM((G, H), jnp.float32)],
    )(hs, batch[:, None], gf, wh1[:H], wh1[H : 2 * H], wh1[2 * H :],
      params["bh1"][None, :], params["Wh2"], params["bh2"][None, :])
    return out[:, 0]


def kernel(x, edge_index, batch, global_feats, params):
    e = edge_index.shape[1]
    src, dst = edge_index[0], edge_index[1]
    cdiv = lambda a, b: -(-a // b)

    nc_a = cdiv(cdiv(cdiv(e, NC * NS), K), CB) * CB
    e_pad_a = NC * NS * nc_a * K
    src_a = jnp.concatenate(
        [src, jnp.zeros((e_pad_a - e,), jnp.int32)]).reshape(NC, NS, nc_a, K)
    dst_a = jnp.concatenate(
        [dst, jnp.full((e_pad_a - e,), N, jnp.int32)]).reshape(NC, NS, nc_a, K)

    nc_b = cdiv(cdiv(cdiv(e, NS), K), CB) * CB
    e_pad_b = NS * nc_b * K
    src_b1 = jnp.concatenate(
        [src, jnp.zeros((e_pad_b - e,), jnp.int32)]).reshape(NS, nc_b, K)
    dst_b1 = jnp.concatenate(
        [dst, jnp.full((e_pad_b - e,), N, jnp.int32)]).reshape(NS, nc_b, K)
    src_b = jnp.stack([src_b1, src_b1 + N])
    dst_b = jnp.stack([dst_b1, dst_b1])

    dh = H // 2
    z_a = jnp.zeros((ACC_ROWS, x.shape[1]), jnp.float32)
    z_b = jnp.zeros((ACC_ROWS, dh), jnp.float32)

    layers = params["layers"]
    agg = _sc_segsum(x, src_a, dst_a, z_a, dh=x.shape[1], n_chunks=nc_a)
    hs = _tc_layer(x, agg, layers[0], mode="edge")
    for p in layers[1:]:
        agg = _sc_segsum(hs, src_b, dst_b, z_b, dh=dh, n_chunks=nc_b)
        hs = _tc_layer(hs, agg, p, mode="split")

    return _tc_final(hs, batch, global_feats, params)
